# baseline (device time: 2129536 ns/iter reference)
import jax
import jax.numpy as jnp
from jax import lax
from jax.experimental import pallas as pl
from jax.experimental.pallas import tpu as pltpu


def kernel(x):
    m, n2 = x.shape
    n = n2 // 2

    C = 8
    rows = m // C

    def body(x_ref, out_ref, local_sems, send_sems, recv_sems):
        my_x = lax.axis_index("x")
        my_y = lax.axis_index("y")
        my_z = lax.axis_index("z")
        p = 1 - my_x

        barrier_sem = pltpu.get_barrier_semaphore()
        pl.semaphore_signal(
            barrier_sem, inc=1,
            device_id=(p, my_y, my_z),
            device_id_type=pl.DeviceIdType.MESH,
        )
        pl.semaphore_wait(barrier_sem, 1)

        rdmas = []
        for i in range(C):
            r = pltpu.make_async_remote_copy(
                src_ref=x_ref.at[pl.ds(i * rows, rows), pl.ds(p * n, n)],
                dst_ref=out_ref.at[pl.ds(my_x * m + i * rows, rows), :],
                send_sem=send_sems.at[i],
                recv_sem=recv_sems.at[i],
                device_id=(p, my_y, my_z),
                device_id_type=pl.DeviceIdType.MESH,
            )
            r.start()
            rdmas.append(r)

        locals_ = []
        for i in range(C):
            c = pltpu.make_async_copy(
                x_ref.at[pl.ds(i * rows, rows), pl.ds(my_x * n, n)],
                out_ref.at[pl.ds(my_x * m + i * rows, rows), :],
                local_sems.at[i],
            )
            c.start()
            locals_.append(c)

        for c in locals_:
            c.wait()
        for r in rdmas:
            r.wait()

    return pl.pallas_call(
        body,
        out_shape=jax.ShapeDtypeStruct((2 * m, n), x.dtype),
        in_specs=[pl.BlockSpec(memory_space=pl.ANY)],
        out_specs=pl.BlockSpec(memory_space=pl.ANY),
        scratch_shapes=[
            pltpu.SemaphoreType.DMA((C,)),
            pltpu.SemaphoreType.DMA((C,)),
            pltpu.SemaphoreType.DMA((C,)),
        ],
        compiler_params=pltpu.CompilerParams(collective_id=0),
    )(x)


# device time: 2129487 ns/iter; 1.0000x vs baseline; 1.0000x over previous
import jax
import jax.numpy as jnp
from jax import lax
from jax.experimental import pallas as pl
from jax.experimental.pallas import tpu as pltpu


def kernel(x):
    m, n2 = x.shape
    n = n2 // 2

    C = 8
    rows = m // C

    def body(x_ref, out_ref, local_sems, send_sems, recv_sems):
        my_x = lax.axis_index("x")
        my_y = lax.axis_index("y")
        my_z = lax.axis_index("z")
        p = 1 - my_x

        barrier_sem = pltpu.get_barrier_semaphore()
        pl.semaphore_signal(
            barrier_sem, inc=1,
            device_id=(p, my_y, my_z),
            device_id_type=pl.DeviceIdType.MESH,
        )
        pl.semaphore_wait(barrier_sem, 1)

        rdmas = []
        for i in range(0):
            r = pltpu.make_async_remote_copy(
                src_ref=x_ref.at[pl.ds(i * rows, rows), pl.ds(p * n, n)],
                dst_ref=out_ref.at[pl.ds(my_x * m + i * rows, rows), :],
                send_sem=send_sems.at[i],
                recv_sem=recv_sems.at[i],
                device_id=(p, my_y, my_z),
                device_id_type=pl.DeviceIdType.MESH,
            )
            r.start()
            rdmas.append(r)

        locals_ = []
        for i in range(C):
            c = pltpu.make_async_copy(
                x_ref.at[pl.ds(i * rows, rows), pl.ds(my_x * n, n)],
                out_ref.at[pl.ds(my_x * m + i * rows, rows), :],
                local_sems.at[i],
            )
            c.start()
            locals_.append(c)

        for c in locals_:
            c.wait()
        for r in rdmas:
            r.wait()

    return pl.pallas_call(
        body,
        out_shape=jax.ShapeDtypeStruct((2 * m, n), x.dtype),
        in_specs=[pl.BlockSpec(memory_space=pl.ANY)],
        out_specs=pl.BlockSpec(memory_space=pl.ANY),
        scratch_shapes=[
            pltpu.SemaphoreType.DMA((C,)),
            pltpu.SemaphoreType.DMA((C,)),
            pltpu.SemaphoreType.DMA((C,)),
        ],
        compiler_params=pltpu.CompilerParams(collective_id=0),
    )(x)


# device time: 2039370 ns/iter; 1.0442x vs baseline; 1.0442x over previous
import jax
import jax.numpy as jnp
from jax import lax
from jax.experimental import pallas as pl
from jax.experimental.pallas import tpu as pltpu


def kernel(x):
    m, n2 = x.shape
    n = n2 // 2

    C = 8
    rows = m // C

    def body(x_ref, out_ref, local_sems, send_sems, recv_sems):
        my_x = lax.axis_index("x")
        my_y = lax.axis_index("y")
        my_z = lax.axis_index("z")
        p = 1 - my_x

        if False:
            barrier_sem = pltpu.get_barrier_semaphore()
            pl.semaphore_signal(
                barrier_sem, inc=1,
                device_id=(p, my_y, my_z),
                device_id_type=pl.DeviceIdType.MESH,
            )
            pl.semaphore_wait(barrier_sem, 1)

        rdmas = []
        for i in range(0):
            r = pltpu.make_async_remote_copy(
                src_ref=x_ref.at[pl.ds(i * rows, rows), pl.ds(p * n, n)],
                dst_ref=out_ref.at[pl.ds(my_x * m + i * rows, rows), :],
                send_sem=send_sems.at[i],
                recv_sem=recv_sems.at[i],
                device_id=(p, my_y, my_z),
                device_id_type=pl.DeviceIdType.MESH,
            )
            r.start()
            rdmas.append(r)

        locals_ = []
        for i in range(C):
            c = pltpu.make_async_copy(
                x_ref.at[pl.ds(i * rows, rows), pl.ds(my_x * n, n)],
                out_ref.at[pl.ds(my_x * m + i * rows, rows), :],
                local_sems.at[i],
            )
            c.start()
            locals_.append(c)

        for c in locals_:
            c.wait()
        for r in rdmas:
            r.wait()

    return pl.pallas_call(
        body,
        out_shape=jax.ShapeDtypeStruct((2 * m, n), x.dtype),
        in_specs=[pl.BlockSpec(memory_space=pl.ANY)],
        out_specs=pl.BlockSpec(memory_space=pl.ANY),
        scratch_shapes=[
            pltpu.SemaphoreType.DMA((C,)),
            pltpu.SemaphoreType.DMA((C,)),
            pltpu.SemaphoreType.DMA((C,)),
        ],
    )(x)


# device time: 815198 ns/iter; 2.6123x vs baseline; 2.5017x over previous
import jax
import jax.numpy as jnp
from jax import lax
from jax.experimental import pallas as pl
from jax.experimental.pallas import tpu as pltpu


def kernel(x):
    m, n2 = x.shape
    n = n2 // 2
    C = 16
    R = m // C

    def body(x_ref, out_ref, vin, vkeep, vsend,
             in_sems, local_sems, send_sems, recv_sems):
        my_x = lax.axis_index("x")
        my_y = lax.axis_index("y")
        my_z = lax.axis_index("z")
        p = 1 - my_x

        barrier_sem = pltpu.get_barrier_semaphore()
        pl.semaphore_signal(
            barrier_sem, inc=1,
            device_id=(p, my_y, my_z),
            device_id_type=pl.DeviceIdType.MESH,
        )
        pl.semaphore_wait(barrier_sem, 1)

        def in_dma(i, slot):
            return pltpu.make_async_copy(
                x_ref.at[pl.ds(i * R, R), :], vin.at[slot], in_sems.at[slot]
            )

        in_dma(0, 0).start()
        keeps = []
        rdmas = []
        for i in range(C):
            slot = i % 2
            if i + 1 < C:
                in_dma(i + 1, 1 - slot).start()
            in_dma(i, slot).wait()

            if i >= 2:
                keeps[i - 2].wait()
                rdmas[i - 2].wait_send()

            @pl.when(my_x == 0)
            def _():
                vkeep[slot] = vin[slot, :, :n]
                vsend[slot] = vin[slot, :, n:]

            @pl.when(my_x == 1)
            def _():
                vkeep[slot] = vin[slot, :, n:]
                vsend[slot] = vin[slot, :, :n]

            k = pltpu.make_async_copy(
                vkeep.at[slot],
                out_ref.at[pl.ds(my_x * m + i * R, R), :],
                local_sems.at[i],
            )
            k.start()
            keeps.append(k)

            r = pltpu.make_async_remote_copy(
                src_ref=vsend.at[slot],
                dst_ref=out_ref.at[pl.ds(my_x * m + i * R, R), :],
                send_sem=send_sems.at[i],
                recv_sem=recv_sems.at[i],
                device_id=(p, my_y, my_z),
                device_id_type=pl.DeviceIdType.MESH,
            )
            r.start()
            rdmas.append(r)

        keeps[C - 2].wait()
        keeps[C - 1].wait()
        rdmas[C - 2].wait_send()
        rdmas[C - 1].wait_send()
        for r in rdmas:
            r.wait_recv()

    return pl.pallas_call(
        body,
        out_shape=jax.ShapeDtypeStruct((2 * m, n), x.dtype),
        in_specs=[pl.BlockSpec(memory_space=pl.ANY)],
        out_specs=pl.BlockSpec(memory_space=pl.ANY),
        scratch_shapes=[
            pltpu.VMEM((2, R, n2), jnp.float32),
            pltpu.VMEM((2, R, n), jnp.float32),
            pltpu.VMEM((2, R, n), jnp.float32),
            pltpu.SemaphoreType.DMA((2,)),
            pltpu.SemaphoreType.DMA((C,)),
            pltpu.SemaphoreType.DMA((C,)),
            pltpu.SemaphoreType.DMA((C,)),
        ],
        compiler_params=pltpu.CompilerParams(collective_id=0),
    )(x)
